# baseline (speedup 1.0000x reference)
import functools

import jax
import jax.numpy as jnp
from jax import lax
from jax.experimental import pallas as pl
from jax.experimental.pallas import tpu as pltpu
from jax.experimental.pallas import tpu_sc as plsc


def _build_gather(B, C, K, H, W, fire):
    info = plsc.get_sparse_core_info()
    nc, ns = info.num_cores, info.num_subcores
    nw = nc * ns
    assert B == nw and K % fire == 0

    mesh = plsc.VectorSubcoreMesh(core_axis_name="c", subcore_axis_name="s")

    @functools.partial(
        pl.kernel,
        mesh=mesh,
        out_type=jax.ShapeDtypeStruct((B, K, H, W), jnp.float32),
        scratch_types=[
            pltpu.VMEM((K,), jnp.int32),
            pltpu.SemaphoreType.DMA,
        ],
    )
    def gather_blocks(x_hbm, keep_hbm, out_hbm, keep_v, sem):
        wid = lax.axis_index("s") * nc + lax.axis_index("c")
        b = wid
        pltpu.sync_copy(keep_hbm, keep_v)

        def step(g):
            kvec = keep_v[pl.ds(g * fire, fire)]
            handles = []
            for j in range(fire):
                c = g * fire + j
                ch = kvec[j]
                handles.append(
                    pltpu.async_copy(x_hbm.at[b, ch], out_hbm.at[b, c], sem)
                )
            for h in handles:
                h.wait()

        pl.loop(0, K // fire)(step)

    return gather_blocks


def kernel(x, keep):
    B, C, H, W = x.shape
    K = keep.shape[0]
    gather_blocks = _build_gather(B, C, K, H, W, fire=16)
    return gather_blocks(x, keep)


# speedup vs baseline: 9.6341x; 9.6341x over previous
import functools

import jax
import jax.numpy as jnp
from jax import lax
from jax.experimental import pallas as pl
from jax.experimental.pallas import tpu as pltpu
from jax.experimental.pallas import tpu_sc as plsc

_LANES = 16


def _build_gather(B, C, K, H, W, nbuf):
    info = plsc.get_sparse_core_info()
    nc, ns = info.num_cores, info.num_subcores
    nw = nc * ns
    assert B == nw and K % nbuf == 0

    mesh = plsc.VectorSubcoreMesh(core_axis_name="c", subcore_axis_name="s")

    @functools.partial(
        pl.kernel,
        mesh=mesh,
        out_type=jax.ShapeDtypeStruct((B, K, H, W), jnp.float32),
        scratch_types=[
            pltpu.VMEM((K,), jnp.int32),
            pltpu.VMEM((nbuf, H, W), jnp.float32),
            pltpu.SemaphoreType.DMA,
            pltpu.SemaphoreType.DMA,
        ],
    )
    def gather_blocks(x_hbm, keep_hbm, out_hbm, keep_v, bufs, gsem, ssem):
        wid = lax.axis_index("s") * nc + lax.axis_index("c")
        b = wid
        pltpu.sync_copy(keep_hbm, keep_v)

        def step(g):
            base = g * nbuf
            kvec = keep_v[pl.ds(base, nbuf)]
            gh = []
            for j in range(nbuf):
                gh.append(
                    pltpu.async_copy(x_hbm.at[b, kvec[j]], bufs.at[j], gsem)
                )
            sh = []
            for j in range(nbuf):
                gh[j].wait()
                sh.append(
                    pltpu.async_copy(bufs.at[j], out_hbm.at[b, base + j], ssem)
                )
            for h in sh:
                h.wait()

        pl.loop(0, K // nbuf)(step)

    return gather_blocks


def kernel(x, keep):
    B, C, H, W = x.shape
    K = keep.shape[0]
    gather_blocks = _build_gather(B, C, K, H, W, nbuf=_LANES)
    return gather_blocks(x, keep)


# speedup vs baseline: 13.3759x; 1.3884x over previous
import functools

import jax
import jax.numpy as jnp
from jax import lax
from jax.experimental import pallas as pl
from jax.experimental.pallas import tpu as pltpu
from jax.experimental.pallas import tpu_sc as plsc

_LANES = 16


def _build_gather(B, C, K, H, W):
    info = plsc.get_sparse_core_info()
    nc, ns = info.num_cores, info.num_subcores
    nw = nc * ns
    sub = _LANES // 2
    assert B == nw and K % _LANES == 0
    n_groups = K // _LANES

    mesh = plsc.VectorSubcoreMesh(core_axis_name="c", subcore_axis_name="s")

    @functools.partial(
        pl.kernel,
        mesh=mesh,
        out_type=jax.ShapeDtypeStruct((B, K, H, W), jnp.float32),
        scratch_types=[
            pltpu.VMEM((K,), jnp.int32),
            pltpu.VMEM((2, sub, H, W), jnp.float32),
            pltpu.SemaphoreType.DMA,
            pltpu.SemaphoreType.DMA,
        ],
    )
    def gather_blocks(x_hbm, keep_hbm, out_hbm, idx_v, bufs, gsem, ssem):
        wid = lax.axis_index("s") * nc + lax.axis_index("c")
        b = wid
        base_row = b * C
        pltpu.sync_copy(keep_hbm, idx_v)
        for i in range(K // _LANES):
            s = pl.ds(i * _LANES, _LANES)
            idx_v[s] = idx_v[s] + base_row

        def step(g):
            kvec = idx_v[pl.ds(g * _LANES, _LANES)]
            for p in range(2):
                base = g * _LANES + p * sub

                @pl.when(g >= 1)
                def _():
                    pltpu.make_async_copy(
                        bufs.at[p],
                        out_hbm.at[b, pl.ds(base - _LANES, sub)],
                        ssem,
                    ).wait()

                gh = []
                for j in range(sub):
                    gh.append(
                        pltpu.async_copy(
                            x_hbm.at[kvec[p * sub + j]], bufs.at[p, j], gsem
                        )
                    )
                for h in gh:
                    h.wait()
                pltpu.async_copy(
                    bufs.at[p], out_hbm.at[b, pl.ds(base, sub)], ssem
                )

        pl.loop(0, n_groups)(step)

        for p in range(2):
            pltpu.make_async_copy(
                bufs.at[p],
                out_hbm.at[b, pl.ds((n_groups - 1) * _LANES + p * sub, sub)],
                ssem,
            ).wait()

    return gather_blocks


def kernel(x, keep):
    B, C, H, W = x.shape
    K = keep.shape[0]
    x3 = x.reshape(B * C, H, W)
    gather_blocks = _build_gather(B, C, K, H, W)
    return gather_blocks(x3, keep)


# speedup vs baseline: 34.5502x; 2.5830x over previous
import functools

import jax
import jax.numpy as jnp
from jax import lax
from jax.experimental import pallas as pl
from jax.experimental.pallas import tpu as pltpu
from jax.experimental.pallas import tpu_sc as plsc

_LANES = 16


def _build_gather(P, C, K, rows_chunk):
    info = plsc.get_sparse_core_info()
    nc, ns = info.num_cores, info.num_subcores
    nw = nc * ns
    rows_w = P // nw
    n_chunks = rows_w // rows_chunk
    assert P % nw == 0 and rows_w % rows_chunk == 0 and K % _LANES == 0

    mesh = plsc.VectorSubcoreMesh(core_axis_name="c", subcore_axis_name="s")

    @functools.partial(
        pl.kernel,
        mesh=mesh,
        out_type=jax.ShapeDtypeStruct((P, K), jnp.float32),
        compiler_params=pltpu.CompilerParams(needs_layout_passes=False),
        scratch_types=[
            pltpu.VMEM((K,), jnp.int32),
            pltpu.VMEM((2, rows_chunk, C), jnp.float32),
            pltpu.VMEM((2, rows_chunk, K), jnp.float32),
            pltpu.SemaphoreType.DMA,
            pltpu.SemaphoreType.DMA,
        ],
    )
    def gather_rows(x_hbm, keep_hbm, out_hbm, keep_v, ibufs, obufs, isem, osem):
        wid = lax.axis_index("s") * nc + lax.axis_index("c")
        row0 = wid * rows_w
        pltpu.sync_copy(keep_hbm, keep_v)
        kvecs = [keep_v[pl.ds(j * _LANES, _LANES)] for j in range(K // _LANES)]

        def in_slice(k):
            return x_hbm.at[pl.ds(row0 + k * rows_chunk, rows_chunk)]

        def out_slice(k):
            return out_hbm.at[pl.ds(row0 + k * rows_chunk, rows_chunk)]

        pltpu.async_copy(in_slice(0), ibufs.at[0], isem)

        def step(k):
            p = lax.rem(k, 2)

            @pl.when(k + 1 < n_chunks)
            def _():
                pltpu.async_copy(in_slice(k + 1), ibufs.at[1 - p], isem)

            pltpu.make_async_copy(in_slice(k), ibufs.at[p], isem).wait()

            @pl.when(k >= 2)
            def _():
                pltpu.make_async_copy(
                    obufs.at[p], out_slice(k - 2), osem
                ).wait()

            pv = jnp.broadcast_to(p, (_LANES,)).astype(jnp.int32)

            def row(r):
                rv = jnp.broadcast_to(r, (_LANES,)).astype(jnp.int32)
                for j in range(K // _LANES):
                    obufs[p, r, pl.ds(j * _LANES, _LANES)] = plsc.load_gather(
                        ibufs, [pv, rv, kvecs[j]]
                    )

            pl.loop(0, rows_chunk)(row)
            pltpu.async_copy(obufs.at[p], out_slice(k), osem)

        pl.loop(0, n_chunks)(step)

        for k in (n_chunks - 2, n_chunks - 1):
            pltpu.make_async_copy(obufs.at[k % 2], out_slice(k), osem).wait()

    return gather_rows


def kernel(x, keep):
    B, C, H, W = x.shape
    K = keep.shape[0]
    P = B * H * W
    xt = x.transpose(0, 2, 3, 1).reshape(P, C)
    gather_rows = _build_gather(P, C, K, rows_chunk=56)
    out_t = gather_rows(xt, keep)
    return out_t.reshape(B, H, W, K).transpose(0, 3, 1, 2)


# speedup vs baseline: 38.1809x; 1.1051x over previous
import functools

import jax
import jax.numpy as jnp
from jax import lax
from jax.experimental import pallas as pl
from jax.experimental.pallas import tpu as pltpu
from jax.experimental.pallas import tpu_sc as plsc

_LANES = 16


def _build_gather(P, C, K, rows_chunk):
    info = plsc.get_sparse_core_info()
    nc, ns = info.num_cores, info.num_subcores
    nw = nc * ns
    rows_w = P // nw
    n_chunks = rows_w // rows_chunk
    assert P % nw == 0 and rows_w % rows_chunk == 0 and K % _LANES == 0

    mesh = plsc.VectorSubcoreMesh(core_axis_name="c", subcore_axis_name="s")

    @functools.partial(
        pl.kernel,
        mesh=mesh,
        out_type=jax.ShapeDtypeStruct((P, K), jnp.float32),
        compiler_params=pltpu.CompilerParams(needs_layout_passes=False),
        scratch_types=[
            pltpu.VMEM((K,), jnp.int32),
            pltpu.VMEM((2, rows_chunk, C), jnp.float32),
            pltpu.VMEM((2, rows_chunk, K), jnp.float32),
            pltpu.SemaphoreType.DMA,
            pltpu.SemaphoreType.DMA,
        ],
    )
    def gather_rows(x_hbm, keep_hbm, out_hbm, keep_v, ibufs, obufs, isem, osem):
        wid = lax.axis_index("s") * nc + lax.axis_index("c")
        row0 = wid * rows_w
        pltpu.sync_copy(keep_hbm, keep_v)
        kvecs = [keep_v[pl.ds(j * _LANES, _LANES)] for j in range(K // _LANES)]

        def in_slice(k):
            return x_hbm.at[pl.ds(row0 + k * rows_chunk, rows_chunk)]

        def out_slice(k):
            return out_hbm.at[pl.ds(row0 + k * rows_chunk, rows_chunk)]

        pltpu.async_copy(in_slice(0), ibufs.at[0], isem)

        def step(k):
            p = lax.rem(k, 2)

            @pl.when(k + 1 < n_chunks)
            def _():
                pltpu.async_copy(in_slice(k + 1), ibufs.at[1 - p], isem)

            pltpu.make_async_copy(in_slice(k), ibufs.at[p], isem).wait()

            @pl.when(k >= 2)
            def _():
                pltpu.make_async_copy(
                    obufs.at[p], out_slice(k - 2), osem
                ).wait()

            pv = jnp.broadcast_to(p, (_LANES,)).astype(jnp.int32)

            def row(r):
                rv = jnp.broadcast_to(r, (_LANES,)).astype(jnp.int32)
                for j in range(K // _LANES):
                    obufs[p, r, pl.ds(j * _LANES, _LANES)] = plsc.load_gather(
                        ibufs, [pv, rv, kvecs[j]]
                    )

            pl.loop(0, rows_chunk, unroll=4)(row)
            pltpu.async_copy(obufs.at[p], out_slice(k), osem)

        pl.loop(0, n_chunks)(step)

        for k in (n_chunks - 2, n_chunks - 1):
            pltpu.make_async_copy(obufs.at[k % 2], out_slice(k), osem).wait()

    return gather_rows


def kernel(x, keep):
    B, C, H, W = x.shape
    K = keep.shape[0]
    P = B * H * W
    xt = x.transpose(0, 2, 3, 1).reshape(P, C)
    gather_rows = _build_gather(P, C, K, rows_chunk=56)
    out_t = gather_rows(xt, keep)
    return out_t.reshape(B, H, W, K).transpose(0, 3, 1, 2)
